# baseline (speedup 1.0000x reference)
import functools

import jax
import jax.numpy as jnp
from jax import lax
from jax.experimental import pallas as pl
from jax.experimental.pallas import tpu as pltpu
from jax.experimental.pallas import tpu_sc as plsc

VOC = 1_000_000
EMB = 64
B = 16384
K = 20
R = 20
KR = K + R

NC = 2
NS = 16
NW = NC * NS
NB = B // NW
SB = 16
NSUB = NB // SB
TASKS = SB * KR
GCHUNK = 128
NG = TASKS // GCHUNK
NLANE = 16
NV = EMB // NLANE


@functools.partial(
    pl.kernel,
    out_type=jax.ShapeDtypeStruct((B * KR,), jnp.float32),
    mesh=plsc.VectorSubcoreMesh(core_axis_name="c", subcore_axis_name="s"),
    compiler_params=pltpu.CompilerParams(
        needs_layout_passes=False, use_tc_tiling_on_sc=False),
    scratch_types=[
        pltpu.VMEM((NB,), jnp.int32),
        pltpu.VMEM((NB * KR,), jnp.int32),
        pltpu.VMEM((SB, EMB), jnp.float32),
        pltpu.VMEM((TASKS, EMB), jnp.float32),
        pltpu.VMEM((TASKS * NLANE,), jnp.float32),
        pltpu.VMEM((TASKS,), jnp.float32),
        pltpu.SemaphoreType.DMA,
    ],
)
def _sc_dots(center_hbm, cw_hbm, emb_hbm, lw_hbm, dots_hbm,
             cidx, widx, ebuf, wbuf, pbuf, dbuf, sem):
    wid = lax.axis_index("s") * NC + lax.axis_index("c")
    b0 = pl.multiple_of(wid * NB, NB)
    t0 = pl.multiple_of(wid * (NB * KR), NB * KR)
    pltpu.sync_copy(center_hbm.at[pl.ds(b0, NB)], cidx)
    pltpu.sync_copy(cw_hbm.at[pl.ds(t0, NB * KR)], widx)

    lane = lax.iota(jnp.int32, NLANE)

    @pl.loop(0, NSUB)
    def _subchunk(s):
        sb0 = pl.multiple_of(s * SB, SB)
        st0 = pl.multiple_of(s * TASKS, TASKS)
        copies = [pltpu.async_copy(emb_hbm.at[cidx.at[pl.ds(sb0, SB)]],
                                   ebuf, sem)]
        for q in range(NG):
            copies.append(pltpu.async_copy(
                lw_hbm.at[widx.at[pl.ds(st0 + q * GCHUNK, GCHUNK)]],
                wbuf.at[pl.ds(q * GCHUNK, GCHUNK)], sem))
        for c in copies:
            c.wait()

        @pl.loop(0, SB)
        def _per_b(b):
            e = [ebuf[b, pl.ds(j * NLANE, NLANE)] for j in range(NV)]

            @pl.loop(0, KR)
            def _per_k(k):
                t = b * KR + k
                p = wbuf[t, pl.ds(0, NLANE)] * e[0]
                for j in range(1, NV):
                    p = p + wbuf[t, pl.ds(j * NLANE, NLANE)] * e[j]
                pbuf[pl.ds(pl.multiple_of(t * NLANE, NLANE), NLANE)] = p

        @pl.loop(0, TASKS // NLANE)
        def _per_g(g):
            base = g * (NLANE * NLANE) + lane * NLANE
            acc = plsc.load_gather(pbuf, [base])
            for j in range(1, NLANE):
                acc = acc + plsc.load_gather(pbuf, [base + j])
            dbuf[pl.ds(pl.multiple_of(g * NLANE, NLANE), NLANE)] = acc

        pltpu.sync_copy(dbuf, dots_hbm.at[pl.ds(t0 + st0, TASKS)])


def _tc_loss_body(d_ref, o_ref):
    d = d_ref[...]
    col = lax.broadcasted_iota(jnp.int32, (B, KR), 1)
    act = jax.nn.sigmoid(d)
    pos = -jnp.log(act)
    neg = -jnp.log(1.0 - act + 1e-3)
    is_pos = col < K
    s_pos = jnp.sum(jnp.where(is_pos, pos, 0.0))
    s_neg = jnp.sum(jnp.where(is_pos, 0.0, neg))
    o_ref[0, 0] = s_pos / (B * K) + s_neg / (B * R)


_tc_loss = pl.pallas_call(
    _tc_loss_body,
    out_shape=jax.ShapeDtypeStruct((1, 1), jnp.float32),
    out_specs=pl.BlockSpec(memory_space=pltpu.SMEM),
)


def kernel(center, context, rand, embeddings, linear_w):
    center = center.astype(jnp.int32)
    cw = jnp.concatenate([context, rand], axis=1).astype(jnp.int32)
    dots = _sc_dots(center, cw.reshape(-1), embeddings, linear_w)
    loss = _tc_loss(dots.reshape(B, KR))
    return loss[0, 0]
